# baseline (speedup 1.0000x reference)
import functools

import jax
import jax.numpy as jnp
from jax import lax
from jax.experimental import pallas as pl
from jax.experimental.pallas import tpu as pltpu
from jax.experimental.pallas import tpu_sc as plsc

_D = 64
_B = 16384
_LAMBDA = 0.001
_NC, _NS, _L = 2, 16, 16
_NW = _NC * _NS
_BPW = _B // _NW
_CHUNK = 128
_NCH = _BPW // _CHUNK
_SCALE = _LAMBDA / _B

_mesh = plsc.VectorSubcoreMesh(core_axis_name="c", subcore_axis_name="s")


@functools.partial(
    pl.kernel,
    mesh=_mesh,
    out_type=jax.ShapeDtypeStruct((_NW * _L,), jnp.float32),
    compiler_params=pltpu.CompilerParams(use_tc_tiling_on_sc=False),
    scratch_types=[
        pltpu.VMEM((_BPW,), jnp.int32),
        pltpu.VMEM((_NCH, _CHUNK, _D), jnp.float32),
        pltpu.VMEM((_BPW * _D,), jnp.float32),
        pltpu.VMEM((_L,), jnp.float32),
        pltpu.SemaphoreType.DMA,
    ],
)
def _center_loss_sc(feat_hbm, lab_hbm, cent_hbm, out_hbm,
                    lab_v, rows_v, feat_v, out_v, sem):
    wid = lax.axis_index("s") * _NC + lax.axis_index("c")
    base = wid * _BPW
    pltpu.sync_copy(lab_hbm.at[pl.ds(base, _BPW)], lab_v)
    feat_cp = pltpu.async_copy(
        feat_hbm.at[pl.ds(base * _D, _BPW * _D)], feat_v, sem)
    gather_cps = [
        pltpu.async_copy(cent_hbm.at[lab_v.at[pl.ds(ch * _CHUNK, _CHUNK)]],
                         rows_v.at[ch], sem)
        for ch in range(_NCH)
    ]
    feat_cp.wait()
    for cp in gather_cps:
        cp.wait()

    acc = jnp.zeros((_L,), jnp.float32)
    for ch in range(_NCH):
        def body(i, a, ch=ch):
            for d in range(_D // _L):
                f = feat_v[pl.ds((ch * _CHUNK) * _D + i * _D + d * _L, _L)]
                c = rows_v[ch, i, pl.ds(d * _L, _L)]
                df = f - c
                a = a + df * df
            return a
        acc = lax.fori_loop(0, _CHUNK, body, acc)

    out_v[...] = acc * _SCALE
    pltpu.sync_copy(out_v, out_hbm.at[pl.ds(wid * _L, _L)])


def kernel(features, labels, centers):
    feat = features.reshape(_B * _D)
    lab = labels.astype(jnp.int32)
    partials = _center_loss_sc(feat, lab, centers)
    return jnp.sum(partials)


# speedup vs baseline: 1.0554x; 1.0554x over previous
import functools

import jax
import jax.numpy as jnp
from jax import lax
from jax.experimental import pallas as pl
from jax.experimental.pallas import tpu as pltpu
from jax.experimental.pallas import tpu_sc as plsc

_D = 64
_B = 16384
_V = 100000
_LAMBDA = 0.001
_NC, _NS, _L = 2, 16, 16
_NW = _NC * _NS
_DPW = _D // _NW
_LCH = 8192
_NLCH = _B // _LCH
_SCALE = _LAMBDA / _B

_mesh = plsc.VectorSubcoreMesh(core_axis_name="c", subcore_axis_name="s")


@functools.partial(
    pl.kernel,
    mesh=_mesh,
    out_type=jax.ShapeDtypeStruct((_NW * _L,), jnp.float32),
    compiler_params=pltpu.CompilerParams(
        use_tc_tiling_on_sc=False, needs_layout_passes=False),
    scratch_types=[
        pltpu.VMEM((_V,), jnp.float32),
        pltpu.VMEM((_B,), jnp.float32),
        pltpu.VMEM((_LCH,), jnp.int32),
        pltpu.VMEM((_L,), jnp.float32),
        pltpu.SemaphoreType.DMA,
    ],
)
def _center_loss_sc(ft_hbm, lab_hbm, ct_hbm, out_hbm,
                    crow_v, frow_v, lab_v, out_v, sem):
    wid = lax.axis_index("s") * _NC + lax.axis_index("c")

    acc = jnp.zeros((_L,), jnp.float32)
    for rep in range(_DPW):
        d = wid * _DPW + rep
        pltpu.async_copy(ct_hbm.at[d], crow_v, sem).wait()
        pltpu.sync_copy(ft_hbm.at[d], frow_v)
        for h in range(_NLCH):
            pltpu.sync_copy(lab_hbm.at[pl.ds(h * _LCH, _LCH)], lab_v)

            def body(k, a, h=h):
                idx = lab_v[pl.ds(k * _L, _L)]
                c = plsc.load_gather(crow_v, [idx])
                f = frow_v[pl.ds(h * _LCH + k * _L, _L)]
                df = f - c
                return a + df * df

            acc = lax.fori_loop(0, _LCH // _L, body, acc)

    out_v[...] = acc * _SCALE
    pltpu.sync_copy(out_v, out_hbm.at[pl.ds(wid * _L, _L)])


def kernel(features, labels, centers):
    ft = features.T
    ct = centers.T
    lab = labels.astype(jnp.int32)
    partials = _center_loss_sc(ft, lab, ct)
    return jnp.sum(partials)


# speedup vs baseline: 1.8694x; 1.7712x over previous
import functools

import jax
import jax.numpy as jnp
from jax import lax
from jax.experimental import pallas as pl
from jax.experimental.pallas import tpu as pltpu
from jax.experimental.pallas import tpu_sc as plsc

_D = 64
_B = 16384
_V = 100000
_LAMBDA = 0.001
_NC, _NS, _L = 2, 16, 16
_NW = _NC * _NS
_DPW = _D // _NW
_LCH = 8192
_NLCH = _B // _LCH
_SCALE = _LAMBDA / _B

_mesh = plsc.VectorSubcoreMesh(core_axis_name="c", subcore_axis_name="s")


@functools.partial(
    pl.kernel,
    mesh=_mesh,
    out_type=jax.ShapeDtypeStruct((_NW * _L,), jnp.float32),
    compiler_params=pltpu.CompilerParams(
        use_tc_tiling_on_sc=True, needs_layout_passes=False),
    scratch_types=[
        pltpu.VMEM((_V,), jnp.float32),
        pltpu.VMEM((_B,), jnp.float32),
        pltpu.VMEM((_LCH,), jnp.int32),
        pltpu.VMEM((_L,), jnp.float32),
        pltpu.SemaphoreType.DMA,
    ],
)
def _center_loss_sc(ft_hbm, lab_hbm, ct_hbm, out_hbm,
                    crow_v, frow_v, lab_v, out_v, sem):
    wid = lax.axis_index("s") * _NC + lax.axis_index("c")

    acc = jnp.zeros((_L,), jnp.float32)
    for rep in range(_DPW):
        d = wid * _DPW + rep
        pltpu.async_copy(ct_hbm.at[d], crow_v, sem).wait()
        pltpu.sync_copy(ft_hbm.at[d], frow_v)
        for h in range(_NLCH):
            pltpu.sync_copy(lab_hbm.at[pl.ds(h * _LCH, _LCH)], lab_v)

            def body(k, a, h=h):
                idx = lab_v[pl.ds(k * _L, _L)]
                c = plsc.load_gather(crow_v, [idx])
                f = frow_v[pl.ds(h * _LCH + k * _L, _L)]
                df = f - c
                return a + df * df

            acc = lax.fori_loop(0, _LCH // _L, body, acc)

    out_v[...] = acc * _SCALE
    pltpu.sync_copy(out_v, out_hbm.at[pl.ds(wid * _L, _L)])


def kernel(features, labels, centers):
    ft = features.T
    ct = centers.T
    lab = labels.astype(jnp.int32)
    partials = _center_loss_sc(ft, lab, ct)
    return jnp.sum(partials)


# speedup vs baseline: 2.2156x; 1.1852x over previous
import functools

import jax
import jax.numpy as jnp
from jax import lax
from jax.experimental import pallas as pl
from jax.experimental.pallas import tpu as pltpu
from jax.experimental.pallas import tpu_sc as plsc

_D = 64
_B = 16384
_V = 100000
_LAMBDA = 0.001
_NC, _NS, _L = 2, 16, 16
_NW = _NC * _NS
_DPW = _D // _NW
_SCALE = _LAMBDA / _B

_T_OFF = (0, 33408, 66816)
_T_SZ = (33408, 33408, 33152)
_TAIL0 = 99968
_TAIL = 32
_TBUF = 33408
_NT = 3
_NU = _DPW * _NT

_mesh = plsc.VectorSubcoreMesh(core_axis_name="c", subcore_axis_name="s")


@functools.partial(
    pl.kernel,
    mesh=_mesh,
    out_type=jax.ShapeDtypeStruct((_NW * _L,), jnp.float32),
    compiler_params=pltpu.CompilerParams(
        use_tc_tiling_on_sc=True, needs_layout_passes=False),
    scratch_types=[
        pltpu.VMEM((_TBUF,), jnp.float32),
        pltpu.VMEM((_TBUF,), jnp.float32),
        pltpu.VMEM((_TAIL,), jnp.float32),
        pltpu.VMEM((_B,), jnp.float32),
        pltpu.VMEM((_B,), jnp.float32),
        pltpu.VMEM((_B,), jnp.int32),
        pltpu.VMEM((_L,), jnp.float32),
        pltpu.SemaphoreType.DMA,
        pltpu.SemaphoreType.DMA,
    ],
)
def _center_loss_sc(ft_hbm, lab_hbm, ct_hbm, out_hbm,
                    crow_a, crow_b, tail_v, frow_0, frow_1, lab_v, out_v,
                    csem, fsem):
    wid = lax.axis_index("s") * _NC + lax.axis_index("c")
    crows = (crow_a, crow_b)
    frows = (frow_0, frow_1)

    def crow_copy(u):
        d = wid * _DPW + u // _NT
        t = u % _NT
        return pltpu.async_copy(
            ct_hbm.at[d, pl.ds(_T_OFF[t], _T_SZ[t])],
            crows[u % 2].at[pl.ds(0, _T_SZ[t])], csem)

    lab_cp = pltpu.async_copy(lab_hbm, lab_v, fsem)
    f0_cp = pltpu.async_copy(ft_hbm.at[wid * _DPW], frow_0, fsem)
    cps = {0: crow_copy(0)}
    lab_cp.wait()
    f0_cp.wait()

    accs = tuple(jnp.zeros((_L,), jnp.float32) for _ in range(4))
    f1_cp = None
    for u in range(_NU):
        cps[u].wait()
        if u + 1 < _NU:
            cps[u + 1] = crow_copy(u + 1)
        if u == _NT - 2:
            f1_cp = pltpu.async_copy(ft_hbm.at[wid * _DPW + 1], frow_1, fsem)
        if u == _NT:
            f1_cp.wait()
        if u % _NT == _NT - 1:
            d = wid * _DPW + u // _NT
            pltpu.sync_copy(ct_hbm.at[d, pl.ds(_TAIL0, _TAIL)], tail_v)

        t = u % _NT
        c0 = _T_OFF[t]
        sz = _T_SZ[t]
        crow = crows[u % 2]
        frow = frows[u // _NT]

        last = t == _NT - 1

        @plsc.parallel_loop(0, _B // _L, 4, unroll=2, carry=accs)
        def body(k, a, c0=c0, sz=sz, crow=crow, frow=frow, last=last):
            out = list(a)
            for j in range(4):
                base = (k + j) * _L
                idx = lab_v[pl.ds(base, _L)]
                f = frow[pl.ds(base, _L)]
                inb = (idx >= c0) & (idx < c0 + sz)
                c = plsc.load_gather(crow, [idx - c0], mask=inb)
                df = jnp.where(inb, f - c, 0.0)
                acc_j = out[j] + df * df
                if last:
                    inb2 = idx >= _TAIL0
                    c2 = plsc.load_gather(tail_v, [idx - _TAIL0], mask=inb2)
                    df2 = jnp.where(inb2, f - c2, 0.0)
                    acc_j = acc_j + df2 * df2
                out[j] = acc_j
            return tuple(out)

        accs = body

    acc = (accs[0] + accs[1]) + (accs[2] + accs[3])
    out_v[...] = acc * _SCALE
    pltpu.sync_copy(out_v, out_hbm.at[pl.ds(wid * _L, _L)])


def kernel(features, labels, centers):
    ft = features.T
    ct = centers.T
    lab = labels.astype(jnp.int32)
    partials = _center_loss_sc(ft, lab, ct)
    return jnp.sum(partials)
